# baseline (speedup 1.0000x reference)
import jax
import jax.numpy as jnp
from jax import lax
from jax.experimental import pallas as pl
from jax.experimental.pallas import tpu as pltpu
from jax.experimental.pallas import tpu_sc as plsc

S = 2048
H = 16
LP = 6144
NC = 2
TI = S // 8
TJ = S // 128


def _bias_body(table_hbm, out_hbm, b2_ref, sem):
    c = lax.axis_index("c")
    s = lax.axis_index("s")
    w = s * NC + c
    h = w // 2
    base_i = (w % 2) * (TI // 2)

    for sh in range(8):
        pltpu.sync_copy(table_hbm.at[pl.ds((h * 8 + sh) * LP, LP)],
                        b2_ref.at[sh])

    LAG = 8
    NW = TI // 2

    def row_dmas(v):
        ti = base_i + v
        base0 = pl.multiple_of((4095 - 7) - 8 * ti, 8)
        return [pltpu.make_async_copy(
                    b2_ref.at[:, pl.ds(base0 + 128 * tj, 128)],
                    out_hbm.at[h, ti, tj], sem)
                for tj in range(TJ)]

    def wave(v, carry):
        @pl.when(v < NW)
        def _():
            for d in row_dmas(v):
                d.start()

        @pl.when(v >= LAG)
        def _():
            for d in row_dmas(v - LAG):
                d.wait()
        return carry

    lax.fori_loop(0, NW + LAG, wave, 0)


def kernel(seq_len, bias_table):
    del seq_len
    idx = (7 - jnp.arange(8)[:, None]) + jnp.arange(LP)[None, :]
    tab = jnp.transpose(bias_table[idx], (2, 0, 1)).reshape(H * 8 * LP)
    run = pl.kernel(
        _bias_body,
        out_type=jax.ShapeDtypeStruct((H, TI, TJ, 8, 128), jnp.float32),
        mesh=plsc.VectorSubcoreMesh(core_axis_name="c", subcore_axis_name="s"),
        scratch_types=[pltpu.VMEM((8, LP), jnp.float32),
                       pltpu.SemaphoreType.DMA],
        compiler_params=pltpu.CompilerParams(use_tc_tiling_on_sc=False),
    )
    out5 = run(tab)
    return out5.transpose(0, 1, 3, 2, 4).reshape(1, H, S, S)


# speedup vs baseline: 2.5445x; 2.5445x over previous
import jax
import jax.numpy as jnp
from jax import lax
from jax.experimental import pallas as pl
from jax.experimental.pallas import tpu as pltpu
from jax.experimental.pallas import tpu_sc as plsc

S = 2048
H = 16
LP = 4096
NC = 2
TI = S // 8
TJ = S // 128


def _bias_body(table_hbm, out_hbm, b2_ref, sem):
    c = lax.axis_index("c")
    s = lax.axis_index("s")
    w = s * NC + c
    h = w // 2
    base_i = (w % 2) * (TI // 2)

    for sh in range(8):
        pltpu.sync_copy(table_hbm.at[pl.ds((h * 8 + sh) * LP, LP)],
                        b2_ref.at[sh])

    LAG = 8
    NW = TI // 2

    def row_dmas(v):
        ti = base_i + v
        base0 = pl.multiple_of((4095 - 7 - 2048) - 8 * ti, 8)
        return [pltpu.make_async_copy(
                    b2_ref.at[:, pl.ds(base0 + 128 * tj, 128)],
                    out_hbm.at[h, ti, tj], sem)
                for tj in range(TJ)]

    def wave(v, carry):
        @pl.when(v < NW)
        def _():
            for d in row_dmas(v):
                d.start()

        @pl.when(v >= LAG)
        def _():
            for d in row_dmas(v - LAG):
                d.wait()
        return carry

    lax.fori_loop(0, NW + LAG, wave, 0)


def kernel(seq_len, bias_table):
    del seq_len
    col = jnp.transpose(bias_table)
    tab = jnp.stack([col[:, 2055 - sh:2055 - sh + LP] for sh in range(8)],
                    axis=1)
    tab = tab.reshape(H * 8 * LP)
    run = pl.kernel(
        _bias_body,
        out_type=jax.ShapeDtypeStruct((H, TI, TJ, 8, 128), jnp.float32),
        mesh=plsc.VectorSubcoreMesh(core_axis_name="c", subcore_axis_name="s"),
        scratch_types=[pltpu.VMEM((8, LP), jnp.float32),
                       pltpu.SemaphoreType.DMA],
        compiler_params=pltpu.CompilerParams(use_tc_tiling_on_sc=False),
    )
    out5 = run(tab)
    return out5.transpose(0, 1, 3, 2, 4).reshape(1, H, S, S)


# speedup vs baseline: 2.6335x; 1.0350x over previous
import jax
import jax.numpy as jnp
from jax import lax
from jax.experimental import pallas as pl
from jax.experimental.pallas import tpu as pltpu
from jax.experimental.pallas import tpu_sc as plsc

S = 2048
H = 16
LP = 4096
NC = 2
TI = S // 8
TJ = S // 128


def _bias_body(table_hbm, out_hbm, b2_ref, sem):
    c = lax.axis_index("c")
    s = lax.axis_index("s")
    w = s * NC + c
    h = w // 2
    base_i = (w % 2) * (TI // 2)

    staging = [pltpu.make_async_copy(
                   table_hbm.at[pl.ds((h * 8 + sh) * LP, LP)],
                   b2_ref.at[sh], sem)
               for sh in range(8)]
    for d in staging:
        d.start()
    for d in staging:
        d.wait()

    LAG = 8
    NW = TI // 2

    def row_dmas(v):
        ti = base_i + v
        base0 = pl.multiple_of((4095 - 7 - 2048) - 8 * ti, 8)
        return [pltpu.make_async_copy(
                    b2_ref.at[:, pl.ds(base0 + 128 * tj, 128)],
                    out_hbm.at[h, ti, tj], sem)
                for tj in range(TJ)]

    def wave(v, carry):
        @pl.when(v < NW)
        def _():
            for d in row_dmas(v):
                d.start()

        @pl.when(v >= LAG)
        def _():
            for d in row_dmas(v - LAG):
                d.wait()
        return carry

    lax.fori_loop(0, NW + LAG, wave, 0)


def kernel(seq_len, bias_table):
    del seq_len
    col = jnp.transpose(bias_table)
    tab = jnp.stack([col[:, 2055 - sh:2055 - sh + LP] for sh in range(8)],
                    axis=1)
    tab = tab.reshape(H * 8 * LP)
    run = pl.kernel(
        _bias_body,
        out_type=jax.ShapeDtypeStruct((H, TI, TJ, 8, 128), jnp.float32),
        mesh=plsc.VectorSubcoreMesh(core_axis_name="c", subcore_axis_name="s"),
        scratch_types=[pltpu.VMEM((8, LP), jnp.float32),
                       pltpu.SemaphoreType.DMA],
        compiler_params=pltpu.CompilerParams(use_tc_tiling_on_sc=False),
    )
    out5 = run(tab)
    return out5.transpose(0, 1, 3, 2, 4).reshape(1, H, S, S)
